# baseline (speedup 1.0000x reference)
import functools

import jax
import jax.numpy as jnp
import numpy as np
from jax import lax
from jax.experimental import pallas as pl
from jax.experimental.pallas import tpu as pltpu
from jax.experimental.pallas import tpu_sc as plsc

N_NODES = 10000
NP = 10240
E = 320000
D = 64
NC, NS = 2, 16
NW = NC * NS
CHUNK = 200
NCH = 50
RPS = NP // NS
GB = 80
NQ = NW * GB

_MESH = plsc.VectorSubcoreMesh(core_axis_name="c", subcore_axis_name="s")
_f32 = jnp.float32
_SC_PARAMS = pltpu.CompilerParams(use_tc_tiling_on_sc=False)

_PERM_TAIL = np.arange(NP - N_NODES, dtype=np.int32) % N_NODES
try:
    with jax.default_device(jax.devices("cpu")[0]):
        _PERM_Q = np.concatenate([
            np.asarray(jax.random.permutation(jax.random.key(1), N_NODES),
                       dtype=np.int32),
            _PERM_TAIL,
        ]).reshape(4, NW, GB)
except Exception:
    _PERM_Q = None


def _perm_q():
    if _PERM_Q is not None:
        return jnp.asarray(_PERM_Q)
    perm = jax.random.permutation(jax.random.key(1), N_NODES).astype(jnp.int32)
    return jnp.concatenate([perm, jnp.asarray(_PERM_TAIL)]).reshape(4, NW, GB)


@functools.partial(
    pl.kernel,
    out_type=[jax.ShapeDtypeStruct((NC, NP, D), _f32),
              jax.ShapeDtypeStruct((NQ, D), _f32)],
    mesh=_MESH,
    compiler_params=_SC_PARAMS,
    scratch_types=[
        pltpu.VMEM((NCH, CHUNK), jnp.int32),
        pltpu.VMEM((NCH, CHUNK), jnp.int32),
        pltpu.VMEM((3, CHUNK, D), _f32),
        pltpu.VMEM_SHARED((NP, D), _f32),
        pltpu.SemaphoreType.DMA((3,)),
        pltpu.VMEM((GB,), jnp.int32),
        pltpu.VMEM((GB, D), _f32),
    ],
)
def _sc_scatter(g_hbm, src_hbm, dst_hbm, zeros_hbm, xw_hbm, pidx_hbm,
                out_hbm, xwp_hbm,
                sidx_v, didx_v, rows_v, acc_sp, sem_g, pidx_v, prow_v):
    c = lax.axis_index("c")
    s = lax.axis_index("s")
    wid = c * NS + s
    pltpu.sync_copy(zeros_hbm.at[pl.ds(s * RPS, RPS)],
                    acc_sp.at[pl.ds(s * RPS, RPS)])
    plsc.subcore_barrier()
    pltpu.sync_copy(src_hbm.at[wid], sidx_v)
    pltpu.sync_copy(dst_hbm.at[wid], didx_v)

    def start_gather(j):
        pltpu.async_copy(g_hbm.at[sidx_v.at[j]], rows_v.at[lax.rem(j, 3)],
                         sem_g.at[lax.rem(j, 3)])

    def wait_gather(j):
        pltpu.make_async_copy(g_hbm.at[sidx_v.at[j]],
                              rows_v.at[lax.rem(j, 3)],
                              sem_g.at[lax.rem(j, 3)]).wait()

    start_gather(0)
    start_gather(1)

    pltpu.sync_copy(pidx_hbm.at[wid], pidx_v)
    pltpu.sync_copy(xw_hbm.at[pidx_v], prow_v)
    pltpu.sync_copy(prow_v, xwp_hbm.at[pl.ds(wid * GB, GB)])

    def chunk(j, carry):
        @pl.when(j + 2 < NCH)
        def _():
            start_gather(j + 2)
        wait_gather(j)
        pltpu.sync_copy(rows_v.at[lax.rem(j, 3)],
                        acc_sp.at[didx_v.at[j]], add=True)
        return carry

    lax.fori_loop(0, NCH, chunk, 0)
    plsc.subcore_barrier()
    pltpu.sync_copy(acc_sp.at[pl.ds(s * RPS, RPS)],
                    out_hbm.at[c, pl.ds(s * RPS, RPS)])


@functools.partial(
    pl.kernel,
    out_type=jax.ShapeDtypeStruct((NC, NP, 8), _f32),
    mesh=_MESH,
    compiler_params=_SC_PARAMS,
    scratch_types=[
        pltpu.VMEM((NCH, CHUNK), jnp.int32),
        pltpu.VMEM((CHUNK, 8), _f32),
        pltpu.VMEM_SHARED((NP, 8), _f32),
    ],
)
def _sc_deg(dst_hbm, zeros_hbm, ones_hbm, out_hbm, didx_v, ones_v, acc_sp):
    c = lax.axis_index("c")
    s = lax.axis_index("s")
    wid = c * NS + s
    pltpu.sync_copy(zeros_hbm.at[pl.ds(s * RPS, RPS)],
                    acc_sp.at[pl.ds(s * RPS, RPS)])
    pltpu.sync_copy(ones_hbm, ones_v)
    plsc.subcore_barrier()
    pltpu.sync_copy(dst_hbm.at[wid], didx_v)

    def chunk(j, carry):
        pltpu.sync_copy(ones_v, acc_sp.at[didx_v.at[j]], add=True)
        return carry

    lax.fori_loop(0, NCH, chunk, 0)
    plsc.subcore_barrier()
    pltpu.sync_copy(acc_sp.at[pl.ds(s * RPS, RPS)],
                    out_hbm.at[c, pl.ds(s * RPS, RPS)])


BR = 1280
_GRID = NP // BR


def _dinv(degp):
    return lax.rsqrt(degp[0, :, 0:1] + degp[1, :, 0:1] + 1.0)


def _rowspec(w=D):
    return pl.BlockSpec((BR, w), lambda i: (i, 0))


def _pairspec(w=D):
    return pl.BlockSpec((2, BR, w), lambda i: (0, i, 0))


def _fullspec(a, b):
    return pl.BlockSpec((a, b), lambda i: (0, 0))


def _tc1_body(x_ref, w1_ref, wb_ref, degp_ref, g1_ref, xw_ref):
    i = pl.program_id(0)
    row = i * BR + lax.broadcasted_iota(jnp.int32, (BR, 1), 0)
    live = row < N_NODES
    dinv = jnp.where(live, _dinv(degp_ref[...]), 0.0)
    xx = x_ref[...]
    h2 = jnp.dot(xx, w1_ref[...], preferred_element_type=_f32)
    g1_ref[...] = h2 * dinv
    xw = jnp.dot(xx, wb_ref[...], preferred_element_type=_f32)
    xw_ref[...] = jnp.where(live, xw, 0.0)


_tc1 = pl.pallas_call(
    _tc1_body,
    grid=(_GRID,),
    in_specs=[_rowspec(128), _fullspec(128, D), _fullspec(128, D),
              _pairspec(8)],
    out_specs=[_rowspec(), _rowspec()],
    out_shape=[jax.ShapeDtypeStruct((NP, D), _f32),
               jax.ShapeDtypeStruct((NP, D), _f32)],
)


def _tc_layer_body(sp_ref, g_ref, degp_ref, b_ref, w_ref, out_ref):
    dinv = _dinv(degp_ref[...])
    sp = sp_ref[...]
    agg = dinv * (sp[0] + sp[1] + g_ref[...]) + b_ref[...]
    h = jnp.maximum(agg, 0.0)
    out_ref[...] = jnp.dot(h, w_ref[...], preferred_element_type=_f32) * dinv


_tc_layer = pl.pallas_call(
    _tc_layer_body,
    grid=(_GRID,),
    in_specs=[_pairspec(), _rowspec(), _pairspec(8), _fullspec(1, D),
              _fullspec(D, D)],
    out_specs=_rowspec(),
    out_shape=jax.ShapeDtypeStruct((NP, D), _f32),
)


def _tc_final_body(sp_ref, g_ref, degp_ref, b_ref, xw_ref, xwp_ref, bb_ref,
                   lg_ref, ng_ref):
    dinv = _dinv(degp_ref[...])
    sp = sp_ref[...]
    emb = dinv * (sp[0] + sp[1] + g_ref[...]) + b_ref[...]
    lg_ref[...] = jnp.sum(xw_ref[...] * emb, axis=1, keepdims=True) + bb_ref[0, 0]
    ng_ref[...] = jnp.sum(xwp_ref[...] * emb, axis=1, keepdims=True) + bb_ref[0, 0]


_tc_final = pl.pallas_call(
    _tc_final_body,
    grid=(_GRID,),
    in_specs=[_pairspec(), _rowspec(), _pairspec(8), _fullspec(1, D),
              _rowspec(), _rowspec(), _fullspec(1, 1)],
    out_specs=[_rowspec(1), _rowspec(1)],
    out_shape=[jax.ShapeDtypeStruct((NP, 1), _f32),
               jax.ShapeDtypeStruct((NP, 1), _f32)],
)


def kernel(x, edge_index, W1, b1, W2, b2, W3, b3, W4, b4, Wb, bb):
    src_p = edge_index[0].astype(jnp.int32).reshape(NW, NCH, CHUNK)
    dst_p = edge_index[1].astype(jnp.int32).reshape(NW, NCH, CHUNK)
    zeros64 = jnp.zeros((NP, D), _f32)
    zeros8 = jnp.zeros((NP, 8), _f32)
    ones8 = jnp.ones((CHUNK, 8), _f32)
    perm_q = _perm_q()

    degp = _sc_deg(dst_p, zeros8, ones8)
    g1, xw = _tc1(x, W1, Wb[0], degp)
    S1, xq0 = _sc_scatter(g1, src_p, dst_p, zeros64, xw, perm_q[0])
    g2 = _tc_layer(S1, g1, degp, b1.reshape(1, D), W2)
    S2, xq1 = _sc_scatter(g2, src_p, dst_p, zeros64, xw, perm_q[1])
    g3 = _tc_layer(S2, g2, degp, b2.reshape(1, D), W3)
    S3, xq2 = _sc_scatter(g3, src_p, dst_p, zeros64, xw, perm_q[2])
    g4 = _tc_layer(S3, g3, degp, b3.reshape(1, D), W4)
    S4, xq3 = _sc_scatter(g4, src_p, dst_p, zeros64, xw, perm_q[3])
    xwp = jnp.concatenate([xq0, xq1, xq2, xq3], axis=0)
    lg, ng = _tc_final(S4, g4, degp, b4.reshape(1, D), xw, xwp,
                       bb.reshape(1, 1))
    return lg[:N_NODES, 0], ng[:N_NODES, 0]


# speedup vs baseline: 1.0422x; 1.0422x over previous
import functools

import jax
import jax.numpy as jnp
import numpy as np
from jax import lax
from jax.experimental import pallas as pl
from jax.experimental.pallas import tpu as pltpu
from jax.experimental.pallas import tpu_sc as plsc

N_NODES = 10000
NP = 10240
E = 320000
D = 64
NC, NS = 2, 16
NW = NC * NS
CHUNK = 200
NCH = 50
RPS = NP // NS
GB = 80
NQ = NW * GB

_MESH = plsc.VectorSubcoreMesh(core_axis_name="c", subcore_axis_name="s")
_f32 = jnp.float32
_SC_PARAMS = pltpu.CompilerParams(use_tc_tiling_on_sc=False)

_PERM_TAIL = np.arange(NP - N_NODES, dtype=np.int32) % N_NODES
try:
    with jax.default_device(jax.devices("cpu")[0]):
        _PERM_Q = np.concatenate([
            np.asarray(jax.random.permutation(jax.random.key(1), N_NODES),
                       dtype=np.int32),
            _PERM_TAIL,
        ]).reshape(4, NW, GB)
except Exception:
    _PERM_Q = None


def _perm_q():
    if _PERM_Q is not None:
        return jnp.asarray(_PERM_Q)
    perm = jax.random.permutation(jax.random.key(1), N_NODES).astype(jnp.int32)
    return jnp.concatenate([perm, jnp.asarray(_PERM_TAIL)]).reshape(4, NW, GB)


@functools.partial(
    pl.kernel,
    out_type=[jax.ShapeDtypeStruct((NC, NP, D), _f32),
              jax.ShapeDtypeStruct((NQ, D), _f32)],
    mesh=_MESH,
    compiler_params=_SC_PARAMS,
    scratch_types=[
        pltpu.VMEM((NCH, CHUNK), jnp.int32),
        pltpu.VMEM((NCH, CHUNK), jnp.int32),
        pltpu.VMEM((3, CHUNK, D), _f32),
        pltpu.VMEM_SHARED((NP, D), _f32),
        pltpu.SemaphoreType.DMA((3,)),
        pltpu.VMEM((GB,), jnp.int32),
        pltpu.VMEM((GB, D), _f32),
    ],
)
def _sc_scatter(g_hbm, ei_hbm, zeros_hbm, xw_hbm, pidx_hbm,
                out_hbm, xwp_hbm,
                sidx_v, didx_v, rows_v, acc_sp, sem_g, pidx_v, prow_v):
    c = lax.axis_index("c")
    s = lax.axis_index("s")
    wid = c * NS + s
    pltpu.sync_copy(zeros_hbm.at[pl.ds(s * RPS, RPS)],
                    acc_sp.at[pl.ds(s * RPS, RPS)])
    plsc.subcore_barrier()
    pltpu.sync_copy(ei_hbm.at[0, wid], sidx_v)
    pltpu.sync_copy(ei_hbm.at[1, wid], didx_v)

    def start_gather(j):
        pltpu.async_copy(g_hbm.at[sidx_v.at[j]], rows_v.at[lax.rem(j, 3)],
                         sem_g.at[lax.rem(j, 3)])

    def wait_gather(j):
        pltpu.make_async_copy(g_hbm.at[sidx_v.at[j]],
                              rows_v.at[lax.rem(j, 3)],
                              sem_g.at[lax.rem(j, 3)]).wait()

    start_gather(0)
    start_gather(1)

    pltpu.sync_copy(pidx_hbm.at[wid], pidx_v)
    pltpu.sync_copy(xw_hbm.at[pidx_v], prow_v)
    pltpu.sync_copy(prow_v, xwp_hbm.at[pl.ds(wid * GB, GB)])

    def chunk(j, carry):
        @pl.when(j + 2 < NCH)
        def _():
            start_gather(j + 2)
        wait_gather(j)
        pltpu.sync_copy(rows_v.at[lax.rem(j, 3)],
                        acc_sp.at[didx_v.at[j]], add=True)
        return carry

    lax.fori_loop(0, NCH, chunk, 0)
    plsc.subcore_barrier()
    pltpu.sync_copy(acc_sp.at[pl.ds(s * RPS, RPS)],
                    out_hbm.at[c, pl.ds(s * RPS, RPS)])


@functools.partial(
    pl.kernel,
    out_type=jax.ShapeDtypeStruct((NC, NP, 8), _f32),
    mesh=_MESH,
    compiler_params=_SC_PARAMS,
    scratch_types=[
        pltpu.VMEM((NCH, CHUNK), jnp.int32),
        pltpu.VMEM((CHUNK, 8), _f32),
        pltpu.VMEM_SHARED((NP, 8), _f32),
    ],
)
def _sc_deg(ei_hbm, zeros_hbm, ones_hbm, out_hbm, didx_v, ones_v, acc_sp):
    c = lax.axis_index("c")
    s = lax.axis_index("s")
    wid = c * NS + s
    pltpu.sync_copy(zeros_hbm.at[pl.ds(s * RPS, RPS)],
                    acc_sp.at[pl.ds(s * RPS, RPS)])
    pltpu.sync_copy(ones_hbm, ones_v)
    plsc.subcore_barrier()
    pltpu.sync_copy(ei_hbm.at[1, wid], didx_v)

    def chunk(j, carry):
        pltpu.sync_copy(ones_v, acc_sp.at[didx_v.at[j]], add=True)
        return carry

    lax.fori_loop(0, NCH, chunk, 0)
    plsc.subcore_barrier()
    pltpu.sync_copy(acc_sp.at[pl.ds(s * RPS, RPS)],
                    out_hbm.at[c, pl.ds(s * RPS, RPS)])


def _dinv(degp):
    return lax.rsqrt(degp[0, :, 0:1] + degp[1, :, 0:1] + 1.0)


def _tc1_body(x_ref, w1_ref, wb_ref, degp_ref, g1_ref, xw_ref):
    dinv = _dinv(degp_ref[...])[:N_NODES]
    xx = x_ref[...]
    h2 = jnp.dot(xx, w1_ref[...], preferred_element_type=_f32)
    g1_ref[:N_NODES] = h2 * dinv
    g1_ref[N_NODES:] = jnp.zeros((NP - N_NODES, D), _f32)
    xw_ref[:N_NODES] = jnp.dot(xx, wb_ref[...], preferred_element_type=_f32)
    xw_ref[N_NODES:] = jnp.zeros((NP - N_NODES, D), _f32)


_tc1 = pl.pallas_call(
    _tc1_body,
    out_shape=[jax.ShapeDtypeStruct((NP, D), _f32),
               jax.ShapeDtypeStruct((NP, D), _f32)],
)


def _tc_layer_body(sp_ref, g_ref, degp_ref, b_ref, w_ref, out_ref):
    dinv = _dinv(degp_ref[...])
    sp = sp_ref[...]
    agg = dinv * (sp[0] + sp[1] + g_ref[...]) + b_ref[...]
    h = jnp.maximum(agg, 0.0)
    out_ref[...] = jnp.dot(h, w_ref[...], preferred_element_type=_f32) * dinv


_tc_layer = pl.pallas_call(
    _tc_layer_body,
    out_shape=jax.ShapeDtypeStruct((NP, D), _f32),
)


def _tc_final_body(sp_ref, g_ref, degp_ref, b_ref, xw_ref,
                   xq0_ref, xq1_ref, xq2_ref, xq3_ref, bb_ref,
                   lg_ref, ng_ref):
    dinv = _dinv(degp_ref[...])
    sp = sp_ref[...]
    emb = dinv * (sp[0] + sp[1] + g_ref[...]) + b_ref[...]
    xwp = jnp.concatenate(
        [xq0_ref[...], xq1_ref[...], xq2_ref[...], xq3_ref[...]], axis=0)
    lg_ref[...] = jnp.sum(xw_ref[...] * emb, axis=1, keepdims=True) + bb_ref[0, 0]
    ng_ref[...] = jnp.sum(xwp * emb, axis=1, keepdims=True) + bb_ref[0, 0]


_tc_final = pl.pallas_call(
    _tc_final_body,
    out_shape=[jax.ShapeDtypeStruct((NP, 1), _f32),
               jax.ShapeDtypeStruct((NP, 1), _f32)],
)


def kernel(x, edge_index, W1, b1, W2, b2, W3, b3, W4, b4, Wb, bb):
    ei_p = edge_index.astype(jnp.int32).reshape(2, NW, NCH, CHUNK)
    zeros64 = jnp.zeros((NP, D), _f32)
    zeros8 = jnp.zeros((NP, 8), _f32)
    ones8 = jnp.ones((CHUNK, 8), _f32)
    perm_q = _perm_q()

    degp = _sc_deg(ei_p, zeros8, ones8)
    g1, xw = _tc1(x, W1, Wb[0], degp)
    S1, xq0 = _sc_scatter(g1, ei_p, zeros64, xw, perm_q[0])
    g2 = _tc_layer(S1, g1, degp, b1.reshape(1, D), W2)
    S2, xq1 = _sc_scatter(g2, ei_p, zeros64, xw, perm_q[1])
    g3 = _tc_layer(S2, g2, degp, b2.reshape(1, D), W3)
    S3, xq2 = _sc_scatter(g3, ei_p, zeros64, xw, perm_q[2])
    g4 = _tc_layer(S3, g3, degp, b3.reshape(1, D), W4)
    S4, xq3 = _sc_scatter(g4, ei_p, zeros64, xw, perm_q[3])
    lg, ng = _tc_final(S4, g4, degp, b4.reshape(1, D), xw,
                       xq0, xq1, xq2, xq3, bb.reshape(1, 1))
    return lg[:N_NODES, 0], ng[:N_NODES, 0]


# speedup vs baseline: 1.0445x; 1.0023x over previous
import functools

import jax
import jax.numpy as jnp
import numpy as np
from jax import lax
from jax.experimental import pallas as pl
from jax.experimental.pallas import tpu as pltpu
from jax.experimental.pallas import tpu_sc as plsc

N_NODES = 10000
NP = 10240
E = 320000
D = 64
NC, NS = 2, 16
NW = NC * NS
CHUNK = 200
NCH = 50
RPS = NP // NS
GB = 80
NQ = NW * GB

_MESH = plsc.VectorSubcoreMesh(core_axis_name="c", subcore_axis_name="s")
_f32 = jnp.float32
_SC_PARAMS = pltpu.CompilerParams(use_tc_tiling_on_sc=False)

_PERM_TAIL = np.arange(NP - N_NODES, dtype=np.int32) % N_NODES
try:
    with jax.default_device(jax.devices("cpu")[0]):
        _PERM_Q = np.concatenate([
            np.asarray(jax.random.permutation(jax.random.key(1), N_NODES),
                       dtype=np.int32),
            _PERM_TAIL,
        ]).reshape(4, NW, GB)
except Exception:
    _PERM_Q = None


def _perm_q():
    if _PERM_Q is not None:
        return jnp.asarray(_PERM_Q)
    perm = jax.random.permutation(jax.random.key(1), N_NODES).astype(jnp.int32)
    return jnp.concatenate([perm, jnp.asarray(_PERM_TAIL)]).reshape(4, NW, GB)


@functools.partial(
    pl.kernel,
    out_type=[jax.ShapeDtypeStruct((NC, NP, D), _f32),
              jax.ShapeDtypeStruct((NQ, D), _f32)],
    mesh=_MESH,
    compiler_params=_SC_PARAMS,
    scratch_types=[
        pltpu.VMEM((NCH, CHUNK), jnp.int32),
        pltpu.VMEM((NCH, CHUNK), jnp.int32),
        pltpu.VMEM((4, CHUNK, D), _f32),
        pltpu.VMEM_SHARED((NP, D), _f32),
        pltpu.SemaphoreType.DMA((4,)),
        pltpu.VMEM((GB,), jnp.int32),
        pltpu.VMEM((GB, D), _f32),
    ],
)
def _sc_scatter(g_hbm, ei_hbm, zeros_hbm, xw_hbm, pidx_hbm,
                out_hbm, xwp_hbm,
                sidx_v, didx_v, rows_v, acc_sp, sem_g, pidx_v, prow_v):
    c = lax.axis_index("c")
    s = lax.axis_index("s")
    wid = c * NS + s
    pltpu.sync_copy(zeros_hbm.at[pl.ds(s * RPS, RPS)],
                    acc_sp.at[pl.ds(s * RPS, RPS)])
    plsc.subcore_barrier()
    pltpu.sync_copy(ei_hbm.at[0, wid], sidx_v)
    pltpu.sync_copy(ei_hbm.at[1, wid], didx_v)

    def start_gather(j):
        pltpu.async_copy(g_hbm.at[sidx_v.at[j]], rows_v.at[lax.rem(j, 4)],
                         sem_g.at[lax.rem(j, 4)])

    def wait_gather(j):
        pltpu.make_async_copy(g_hbm.at[sidx_v.at[j]],
                              rows_v.at[lax.rem(j, 4)],
                              sem_g.at[lax.rem(j, 4)]).wait()

    start_gather(0)
    start_gather(1)
    start_gather(2)

    pltpu.sync_copy(pidx_hbm.at[wid], pidx_v)
    pltpu.sync_copy(xw_hbm.at[pidx_v], prow_v)
    pltpu.sync_copy(prow_v, xwp_hbm.at[pl.ds(wid * GB, GB)])

    def chunk(j, carry):
        @pl.when(j + 3 < NCH)
        def _():
            start_gather(j + 3)
        wait_gather(j)
        pltpu.sync_copy(rows_v.at[lax.rem(j, 4)],
                        acc_sp.at[didx_v.at[j]], add=True)
        return carry

    lax.fori_loop(0, NCH, chunk, 0)
    plsc.subcore_barrier()
    pltpu.sync_copy(acc_sp.at[pl.ds(s * RPS, RPS)],
                    out_hbm.at[c, pl.ds(s * RPS, RPS)])


@functools.partial(
    pl.kernel,
    out_type=jax.ShapeDtypeStruct((NC, NP, 8), _f32),
    mesh=_MESH,
    compiler_params=_SC_PARAMS,
    scratch_types=[
        pltpu.VMEM((NCH, CHUNK), jnp.int32),
        pltpu.VMEM((CHUNK, 8), _f32),
        pltpu.VMEM_SHARED((NP, 8), _f32),
    ],
)
def _sc_deg(ei_hbm, zeros_hbm, ones_hbm, out_hbm, didx_v, ones_v, acc_sp):
    c = lax.axis_index("c")
    s = lax.axis_index("s")
    wid = c * NS + s
    pltpu.sync_copy(zeros_hbm.at[pl.ds(s * RPS, RPS)],
                    acc_sp.at[pl.ds(s * RPS, RPS)])
    pltpu.sync_copy(ones_hbm, ones_v)
    plsc.subcore_barrier()
    pltpu.sync_copy(ei_hbm.at[1, wid], didx_v)

    def chunk(j, carry):
        pltpu.sync_copy(ones_v, acc_sp.at[didx_v.at[j]], add=True)
        return carry

    lax.fori_loop(0, NCH, chunk, 0)
    plsc.subcore_barrier()
    pltpu.sync_copy(acc_sp.at[pl.ds(s * RPS, RPS)],
                    out_hbm.at[c, pl.ds(s * RPS, RPS)])


def _dinv(degp):
    return lax.rsqrt(degp[0, :, 0:1] + degp[1, :, 0:1] + 1.0)


def _tc1_body(x_ref, w1_ref, wb_ref, degp_ref, g1_ref, xw_ref):
    dinv = _dinv(degp_ref[...])[:N_NODES]
    xx = x_ref[...]
    h2 = jnp.dot(xx, w1_ref[...], preferred_element_type=_f32)
    g1_ref[:N_NODES] = h2 * dinv
    g1_ref[N_NODES:] = jnp.zeros((NP - N_NODES, D), _f32)
    xw_ref[:N_NODES] = jnp.dot(xx, wb_ref[...], preferred_element_type=_f32)
    xw_ref[N_NODES:] = jnp.zeros((NP - N_NODES, D), _f32)


_tc1 = pl.pallas_call(
    _tc1_body,
    out_shape=[jax.ShapeDtypeStruct((NP, D), _f32),
               jax.ShapeDtypeStruct((NP, D), _f32)],
)


def _tc_layer_body(sp_ref, g_ref, degp_ref, b_ref, w_ref, out_ref):
    dinv = _dinv(degp_ref[...])
    sp = sp_ref[...]
    agg = dinv * (sp[0] + sp[1] + g_ref[...]) + b_ref[...]
    h = jnp.maximum(agg, 0.0)
    out_ref[...] = jnp.dot(h, w_ref[...], preferred_element_type=_f32) * dinv


_tc_layer = pl.pallas_call(
    _tc_layer_body,
    out_shape=jax.ShapeDtypeStruct((NP, D), _f32),
)


def _tc_final_body(sp_ref, g_ref, degp_ref, b_ref, xw_ref,
                   xq0_ref, xq1_ref, xq2_ref, xq3_ref, bb_ref,
                   lg_ref, ng_ref):
    dinv = _dinv(degp_ref[...])
    sp = sp_ref[...]
    emb = dinv * (sp[0] + sp[1] + g_ref[...]) + b_ref[...]
    xwp = jnp.concatenate(
        [xq0_ref[...], xq1_ref[...], xq2_ref[...], xq3_ref[...]], axis=0)
    lg_ref[...] = jnp.sum(xw_ref[...] * emb, axis=1, keepdims=True) + bb_ref[0, 0]
    ng_ref[...] = jnp.sum(xwp * emb, axis=1, keepdims=True) + bb_ref[0, 0]


_tc_final = pl.pallas_call(
    _tc_final_body,
    out_shape=[jax.ShapeDtypeStruct((NP, 1), _f32),
               jax.ShapeDtypeStruct((NP, 1), _f32)],
)


def kernel(x, edge_index, W1, b1, W2, b2, W3, b3, W4, b4, Wb, bb):
    ei_p = edge_index.astype(jnp.int32).reshape(2, NW, NCH, CHUNK)
    zeros64 = jnp.zeros((NP, D), _f32)
    zeros8 = jnp.zeros((NP, 8), _f32)
    ones8 = jnp.ones((CHUNK, 8), _f32)
    perm_q = _perm_q()

    degp = _sc_deg(ei_p, zeros8, ones8)
    g1, xw = _tc1(x, W1, Wb[0], degp)
    S1, xq0 = _sc_scatter(g1, ei_p, zeros64, xw, perm_q[0])
    g2 = _tc_layer(S1, g1, degp, b1.reshape(1, D), W2)
    S2, xq1 = _sc_scatter(g2, ei_p, zeros64, xw, perm_q[1])
    g3 = _tc_layer(S2, g2, degp, b2.reshape(1, D), W3)
    S3, xq2 = _sc_scatter(g3, ei_p, zeros64, xw, perm_q[2])
    g4 = _tc_layer(S3, g3, degp, b3.reshape(1, D), W4)
    S4, xq3 = _sc_scatter(g4, ei_p, zeros64, xw, perm_q[3])
    lg, ng = _tc_final(S4, g4, degp, b4.reshape(1, D), xw,
                       xq0, xq1, xq2, xq3, bb.reshape(1, 1))
    return lg[:N_NODES, 0], ng[:N_NODES, 0]


# speedup vs baseline: 1.0450x; 1.0005x over previous
import functools

import jax
import jax.numpy as jnp
import numpy as np
from jax import lax
from jax.experimental import pallas as pl
from jax.experimental.pallas import tpu as pltpu
from jax.experimental.pallas import tpu_sc as plsc

N_NODES = 10000
NP = 10240
E = 320000
D = 64
NC, NS = 2, 16
NW = NC * NS
CHUNK = 200
NCH = 50
RPS = NP // NS
GB = 80
NQ = NW * GB

_f32 = jnp.float32
_SC_PARAMS = pltpu.CompilerParams(use_tc_tiling_on_sc=False)

_PERM_TAIL = np.arange(NP - N_NODES, dtype=np.int32) % N_NODES
try:
    with jax.default_device(jax.devices("cpu")[0]):
        _PERM_Q = np.concatenate([
            np.asarray(jax.random.permutation(jax.random.key(1), N_NODES),
                       dtype=np.int32),
            _PERM_TAIL,
        ]).reshape(4, NW, GB)
except Exception:
    _PERM_Q = None


def _perm_q():
    if _PERM_Q is not None:
        return jnp.asarray(_PERM_Q)
    perm = jax.random.permutation(jax.random.key(1), N_NODES).astype(jnp.int32)
    return jnp.concatenate([perm, jnp.asarray(_PERM_TAIL)]).reshape(4, NW, GB)


def _sc_scatter_body(g_hbm, ei_hbm, zeros_hbm, xw_hbm, pidx_hbm,
                     out_hbm, xwp_hbm,
                     sidx_v, didx_v, rows_v, acc_sp, sem_g, pidx_v, prow_v):
    c = lax.axis_index("c")
    s = lax.axis_index("s")
    wid = c * NS + s
    pltpu.sync_copy(zeros_hbm.at[pl.ds(s * RPS, RPS)],
                    acc_sp.at[pl.ds(s * RPS, RPS)])
    plsc.subcore_barrier()
    pltpu.sync_copy(ei_hbm.at[0, wid], sidx_v)
    pltpu.sync_copy(ei_hbm.at[1, wid], didx_v)

    def start_gather(j):
        pltpu.async_copy(g_hbm.at[sidx_v.at[j]], rows_v.at[lax.rem(j, 4)],
                         sem_g.at[lax.rem(j, 4)])

    def wait_gather(j):
        pltpu.make_async_copy(g_hbm.at[sidx_v.at[j]],
                              rows_v.at[lax.rem(j, 4)],
                              sem_g.at[lax.rem(j, 4)]).wait()

    start_gather(0)
    start_gather(1)
    start_gather(2)

    pltpu.sync_copy(pidx_hbm.at[wid], pidx_v)
    pltpu.sync_copy(xw_hbm.at[pidx_v], prow_v)
    pltpu.sync_copy(prow_v, xwp_hbm.at[pl.ds(wid * GB, GB)])

    def chunk(j, carry):
        @pl.when(j + 3 < NCH)
        def _():
            start_gather(j + 3)
        wait_gather(j)
        pltpu.sync_copy(rows_v.at[lax.rem(j, 4)],
                        acc_sp.at[didx_v.at[j]], add=True)
        return carry

    lax.fori_loop(0, NCH, chunk, 0)
    plsc.subcore_barrier()
    pltpu.sync_copy(acc_sp.at[pl.ds(s * RPS, RPS)],
                    out_hbm.at[c, pl.ds(s * RPS, RPS)])


def _sc_deg_body(ei_hbm, zeros_hbm, ones_hbm, out_hbm, didx_v, ones_v, acc_sp):
    c = lax.axis_index("c")
    s = lax.axis_index("s")
    wid = c * NS + s
    pltpu.sync_copy(zeros_hbm.at[pl.ds(s * RPS, RPS)],
                    acc_sp.at[pl.ds(s * RPS, RPS)])
    pltpu.sync_copy(ones_hbm, ones_v)
    plsc.subcore_barrier()
    pltpu.sync_copy(ei_hbm.at[1, wid], didx_v)

    def chunk(j, carry):
        pltpu.sync_copy(ones_v, acc_sp.at[didx_v.at[j]], add=True)
        return carry

    lax.fori_loop(0, NCH, chunk, 0)
    plsc.subcore_barrier()
    pltpu.sync_copy(acc_sp.at[pl.ds(s * RPS, RPS)],
                    out_hbm.at[c, pl.ds(s * RPS, RPS)])


@functools.lru_cache(maxsize=None)
def _sc_kernels():
    mesh = plsc.VectorSubcoreMesh(core_axis_name="c", subcore_axis_name="s",
                                  num_cores=NC, num_subcores=NS)
    scatter = pl.kernel(
        _sc_scatter_body,
        out_type=[jax.ShapeDtypeStruct((NC, NP, D), _f32),
                  jax.ShapeDtypeStruct((NQ, D), _f32)],
        mesh=mesh,
        compiler_params=_SC_PARAMS,
        scratch_types=[
            pltpu.VMEM((NCH, CHUNK), jnp.int32),
            pltpu.VMEM((NCH, CHUNK), jnp.int32),
            pltpu.VMEM((4, CHUNK, D), _f32),
            pltpu.VMEM_SHARED((NP, D), _f32),
            pltpu.SemaphoreType.DMA((4,)),
            pltpu.VMEM((GB,), jnp.int32),
            pltpu.VMEM((GB, D), _f32),
        ],
    )
    deg = pl.kernel(
        _sc_deg_body,
        out_type=jax.ShapeDtypeStruct((NC, NP, 8), _f32),
        mesh=mesh,
        compiler_params=_SC_PARAMS,
        scratch_types=[
            pltpu.VMEM((NCH, CHUNK), jnp.int32),
            pltpu.VMEM((CHUNK, 8), _f32),
            pltpu.VMEM_SHARED((NP, 8), _f32),
        ],
    )
    return scatter, deg


def _dinv(degp):
    return lax.rsqrt(degp[0, :, 0:1] + degp[1, :, 0:1] + 1.0)


def _tc1_body(x_ref, w1_ref, wb_ref, degp_ref, g1_ref, xw_ref):
    dinv = _dinv(degp_ref[...])[:N_NODES]
    xx = x_ref[...]
    h2 = jnp.dot(xx, w1_ref[...], preferred_element_type=_f32)
    g1_ref[:N_NODES] = h2 * dinv
    g1_ref[N_NODES:] = jnp.zeros((NP - N_NODES, D), _f32)
    xw_ref[:N_NODES] = jnp.dot(xx, wb_ref[...], preferred_element_type=_f32)
    xw_ref[N_NODES:] = jnp.zeros((NP - N_NODES, D), _f32)


_tc1 = pl.pallas_call(
    _tc1_body,
    out_shape=[jax.ShapeDtypeStruct((NP, D), _f32),
               jax.ShapeDtypeStruct((NP, D), _f32)],
)


def _tc_layer_body(sp_ref, g_ref, degp_ref, b_ref, w_ref, out_ref):
    dinv = _dinv(degp_ref[...])
    sp = sp_ref[...]
    agg = dinv * (sp[0] + sp[1] + g_ref[...]) + b_ref[...]
    h = jnp.maximum(agg, 0.0)
    out_ref[...] = jnp.dot(h, w_ref[...], preferred_element_type=_f32) * dinv


_tc_layer = pl.pallas_call(
    _tc_layer_body,
    out_shape=jax.ShapeDtypeStruct((NP, D), _f32),
)


def _tc_final_body(sp_ref, g_ref, degp_ref, b_ref, xw_ref,
                   xq0_ref, xq1_ref, xq2_ref, xq3_ref, bb_ref,
                   lg_ref, ng_ref):
    dinv = _dinv(degp_ref[...])
    sp = sp_ref[...]
    emb = dinv * (sp[0] + sp[1] + g_ref[...]) + b_ref[...]
    xwp = jnp.concatenate(
        [xq0_ref[...], xq1_ref[...], xq2_ref[...], xq3_ref[...]], axis=0)
    lg_ref[...] = jnp.sum(xw_ref[...] * emb, axis=1, keepdims=True) + bb_ref[0, 0]
    ng_ref[...] = jnp.sum(xwp * emb, axis=1, keepdims=True) + bb_ref[0, 0]


_tc_final = pl.pallas_call(
    _tc_final_body,
    out_shape=[jax.ShapeDtypeStruct((NP, 1), _f32),
               jax.ShapeDtypeStruct((NP, 1), _f32)],
)


def kernel(x, edge_index, W1, b1, W2, b2, W3, b3, W4, b4, Wb, bb):
    ei_p = edge_index.astype(jnp.int32).reshape(2, NW, NCH, CHUNK)
    zeros64 = jnp.zeros((NP, D), _f32)
    zeros8 = jnp.zeros((NP, 8), _f32)
    ones8 = jnp.ones((CHUNK, 8), _f32)
    perm_q = _perm_q()

    _sc_scatter, _sc_deg = _sc_kernels()
    degp = _sc_deg(ei_p, zeros8, ones8)
    g1, xw = _tc1(x, W1, Wb[0], degp)
    S1, xq0 = _sc_scatter(g1, ei_p, zeros64, xw, perm_q[0])
    g2 = _tc_layer(S1, g1, degp, b1.reshape(1, D), W2)
    S2, xq1 = _sc_scatter(g2, ei_p, zeros64, xw, perm_q[1])
    g3 = _tc_layer(S2, g2, degp, b2.reshape(1, D), W3)
    S3, xq2 = _sc_scatter(g3, ei_p, zeros64, xw, perm_q[2])
    g4 = _tc_layer(S3, g3, degp, b3.reshape(1, D), W4)
    S4, xq3 = _sc_scatter(g4, ei_p, zeros64, xw, perm_q[3])
    lg, ng = _tc_final(S4, g4, degp, b4.reshape(1, D), xw,
                       xq0, xq1, xq2, xq3, bb.reshape(1, 1))
    return lg[:N_NODES, 0], ng[:N_NODES, 0]
